# Initial kernel scaffold; baseline (speedup 1.0000x reference)
import jax
import jax.numpy as jnp
from jax.experimental import pallas as pl


def kernel(x, edge_index, edge_weight):
    raise NotImplementedError("write your pallas kernel here")


# speedup vs baseline: 3.3674x; 3.3674x over previous
import functools

import jax
import jax.numpy as jnp
from jax import lax
from jax.experimental import pallas as pl
from jax.experimental.pallas import tpu as pltpu
from jax.experimental.pallas import tpu_sc as plsc

N_NODES = 10000
D_FEAT = 128
N_EDGES = 320000
MIN_NORM = 1e-15
EPS = 4e-3

NC = 2
NS = 16
NW = NC * NS
K = 128
NB = -(-N_EDGES // (NW * K))
EPT = NB * K
E_PAD = EPT * NW
RPT = 632
NP = RPT * NS
DV = D_FEAT // 16


def _spmm_body(table, srcs, dsts, ws, out, accum, idx_s, idx_d, wv, rows, sem):
    c = lax.axis_index("c")
    s = lax.axis_index("s")
    wid = c * NS + s

    zvec = jnp.zeros((16,), jnp.float32)

    def zero_row(i, carry):
        for d in range(DV):
            rows[i, pl.ds(d * 16, 16)] = zvec
        return carry

    lax.fori_loop(0, K, zero_row, 0)
    r0 = s * RPT
    for j in range(RPT // K):
        pltpu.sync_copy(rows, accum.at[pl.ds(r0 + j * K, K)])
    rem = RPT % K
    if rem:
        pltpu.sync_copy(rows.at[pl.ds(0, rem)],
                        accum.at[pl.ds(r0 + (RPT // K) * K, rem)])
    plsc.subcore_barrier()

    base0 = wid * EPT

    def eblock(b, carry):
        base = base0 + b * K
        pltpu.sync_copy(srcs.at[pl.ds(base, K)], idx_s)
        pltpu.sync_copy(dsts.at[pl.ds(base, K)], idx_d)
        pltpu.sync_copy(ws.at[pl.ds(base, K)], wv)
        pltpu.async_copy(table.at[idx_s], rows, sem).wait()

        def scale(g, inner):
            wvec = wv[pl.ds(g * 16, 16)]
            for l in range(16):
                w = wvec[l]
                for d in range(DV):
                    sl = pl.ds(d * 16, 16)
                    rows[g * 16 + l, sl] = rows[g * 16 + l, sl] * w
            return inner

        lax.fori_loop(0, K // 16, scale, 0)
        pltpu.sync_copy(rows, accum.at[idx_d], add=True)
        return carry

    lax.fori_loop(0, NB, eblock, 0)
    plsc.subcore_barrier()

    pltpu.sync_copy(accum.at[pl.ds(r0, RPT)], out.at[c, pl.ds(r0, RPT)])


@functools.cache
def _make_spmm():
    return pl.kernel(
        _spmm_body,
        out_type=jax.ShapeDtypeStruct((NC, NP, D_FEAT), jnp.float32),
        mesh=plsc.VectorSubcoreMesh(core_axis_name="c", subcore_axis_name="s",
                                    num_cores=NC, num_subcores=NS),
        scratch_types=[
            pltpu.VMEM_SHARED((NP, D_FEAT), jnp.float32),
            pltpu.VMEM((K,), jnp.int32),
            pltpu.VMEM((K,), jnp.int32),
            pltpu.VMEM((K,), jnp.float32),
            pltpu.VMEM((K, D_FEAT), jnp.float32),
            pltpu.SemaphoreType.DMA,
        ],
    )


def _logmap0_body(x_ref, o_ref):
    x = x_ref[...]
    norm = jnp.maximum(jnp.sqrt(jnp.sum(x * x, axis=1, keepdims=True)),
                       MIN_NORM)
    z = jnp.clip(norm, -1 + 1e-7, 1 - 1e-7)
    o_ref[...] = (0.5 * jnp.log((1 + z) / (1 - z)) / norm) * x


def _combine_body(p_ref, o_ref):
    o_ref[...] = p_ref[0] + p_ref[1]


def _finish_body(p_ref, o_ref):
    u = p_ref[0] + p_ref[1]
    un = jnp.maximum(jnp.sqrt(jnp.sum(u * u, axis=1, keepdims=True)), MIN_NORM)
    g = jnp.tanh(un) * u / un
    gn = jnp.maximum(jnp.sqrt(jnp.sum(g * g, axis=1, keepdims=True)), MIN_NORM)
    maxnorm = 1.0 - EPS
    o_ref[...] = jnp.where(gn > maxnorm, g / gn * maxnorm, g)


_BR = RPT
_row_spec = pl.BlockSpec((_BR, D_FEAT), lambda i: (i, 0))
_pair_spec = pl.BlockSpec((NC, _BR, D_FEAT), lambda i: (0, i, 0))
_row_shape = jax.ShapeDtypeStruct((NP, D_FEAT), jnp.float32)

_logmap0 = pl.pallas_call(
    _logmap0_body, grid=(NP // _BR,),
    in_specs=[_row_spec], out_specs=_row_spec, out_shape=_row_shape)

_combine = pl.pallas_call(
    _combine_body, grid=(NP // _BR,),
    in_specs=[_pair_spec], out_specs=_row_spec, out_shape=_row_shape)

_finish = pl.pallas_call(
    _finish_body, grid=(NP // _BR,),
    in_specs=[_pair_spec], out_specs=_row_spec, out_shape=_row_shape)


def kernel(x, edge_index, edge_weight):
    src = edge_index[0].astype(jnp.int32)
    dst = edge_index[1].astype(jnp.int32)
    w = edge_weight.astype(jnp.float32)
    epad = E_PAD - N_EDGES
    src = jnp.concatenate([src, jnp.zeros((epad,), jnp.int32)])
    dst = jnp.concatenate([dst, jnp.zeros((epad,), jnp.int32)])
    w = jnp.concatenate([w, jnp.zeros((epad,), jnp.float32)])
    xp = jnp.concatenate(
        [x, jnp.zeros((NP - N_NODES, D_FEAT), jnp.float32)])

    spmm = _make_spmm()
    t = _logmap0(xp)
    p1 = spmm(t, src, dst, w)
    y1 = _combine(p1)
    p2 = spmm(y1, src, dst, w)
    h = _finish(p2)
    return h[:N_NODES]


# speedup vs baseline: 4.4323x; 1.3162x over previous
import functools

import jax
import jax.numpy as jnp
from jax import lax
from jax.experimental import pallas as pl
from jax.experimental.pallas import tpu as pltpu
from jax.experimental.pallas import tpu_sc as plsc

N_NODES = 10000
D_FEAT = 128
N_EDGES = 320000
MIN_NORM = 1e-15
EPS = 4e-3

NC = 2
NS = 16
NW = NC * NS
K = 128
NB = -(-N_EDGES // (NW * K))
EPT = NB * K
E_PAD = EPT * NW
RPT = 632
NP = RPT * NS
DV = D_FEAT // 16


def _spmm_body(table, srcs, dsts, ws, out, accum,
               is0, is1, id0, id1, wv0, wv1, rows0, rows1, gsem0, gsem1):
    c = lax.axis_index("c")
    s = lax.axis_index("s")
    wid = c * NS + s

    zvec = jnp.zeros((16,), jnp.float32)

    def zero_row(i, carry):
        for d in range(DV):
            rows0[i, pl.ds(d * 16, 16)] = zvec
        return carry

    lax.fori_loop(0, K, zero_row, 0)
    r0 = s * RPT
    for j in range(RPT // K):
        pltpu.sync_copy(rows0, accum.at[pl.ds(r0 + j * K, K)])
    rem = RPT % K
    if rem:
        pltpu.sync_copy(rows0.at[pl.ds(0, rem)],
                        accum.at[pl.ds(r0 + (RPT // K) * K, rem)])
    plsc.subcore_barrier()

    base0 = wid * EPT

    def stage(b, isb, idb, wvb):
        base = base0 + b * K
        pltpu.sync_copy(srcs.at[pl.ds(base, K)], isb)
        pltpu.sync_copy(dsts.at[pl.ds(base, K)], idb)
        pltpu.sync_copy(ws.at[pl.ds(base, K)], wvb)

    def gather(isb, rows, sem):
        pltpu.async_copy(table.at[isb], rows, sem)

    def gwait(isb, rows, sem):
        pltpu.make_async_copy(table.at[isb], rows, sem).wait()

    def scale_scatter(idb, wvb, rows):
        def scale(g, inner):
            w16 = wvb[pl.ds(g * 16, 16)]
            for l in range(16):
                wl = w16[l]
                for d in range(DV):
                    sl = pl.ds(d * 16, 16)
                    rows[g * 16 + l, sl] = rows[g * 16 + l, sl] * wl
            return inner

        lax.fori_loop(0, K // 16, scale, 0)
        pltpu.sync_copy(rows, accum.at[idb], add=True)

    assert NB % 2 == 1
    stage(0, is0, id0, wv0)
    gather(is0, rows0, gsem0)

    def pair(i, carry):
        b = 2 * i
        stage(b + 1, is1, id1, wv1)
        gwait(is0, rows0, gsem0)
        gather(is1, rows1, gsem1)
        scale_scatter(id0, wv0, rows0)
        stage(b + 2, is0, id0, wv0)
        gwait(is1, rows1, gsem1)
        gather(is0, rows0, gsem0)
        scale_scatter(id1, wv1, rows1)
        return carry

    lax.fori_loop(0, (NB - 1) // 2, pair, 0)
    gwait(is0, rows0, gsem0)
    scale_scatter(id0, wv0, rows0)
    plsc.subcore_barrier()

    pltpu.sync_copy(accum.at[pl.ds(r0, RPT)], out.at[c, pl.ds(r0, RPT)])


@functools.cache
def _make_spmm():
    return pl.kernel(
        _spmm_body,
        out_type=jax.ShapeDtypeStruct((NC, NP, D_FEAT), jnp.float32),
        mesh=plsc.VectorSubcoreMesh(core_axis_name="c", subcore_axis_name="s",
                                    num_cores=NC, num_subcores=NS),
        scratch_types=[
            pltpu.VMEM_SHARED((NP, D_FEAT), jnp.float32),
            pltpu.VMEM((K,), jnp.int32),
            pltpu.VMEM((K,), jnp.int32),
            pltpu.VMEM((K,), jnp.int32),
            pltpu.VMEM((K,), jnp.int32),
            pltpu.VMEM((K,), jnp.float32),
            pltpu.VMEM((K,), jnp.float32),
            pltpu.VMEM((K, D_FEAT), jnp.float32),
            pltpu.VMEM((K, D_FEAT), jnp.float32),
            pltpu.SemaphoreType.DMA,
            pltpu.SemaphoreType.DMA,
        ],
    )


def _logmap0_body(x_ref, o_ref):
    x = x_ref[...]
    norm = jnp.maximum(jnp.sqrt(jnp.sum(x * x, axis=1, keepdims=True)),
                       MIN_NORM)
    z = jnp.clip(norm, -1 + 1e-7, 1 - 1e-7)
    o_ref[...] = (0.5 * jnp.log((1 + z) / (1 - z)) / norm) * x


def _combine_body(p_ref, o_ref):
    o_ref[...] = p_ref[0] + p_ref[1]


def _finish_body(p_ref, o_ref):
    u = p_ref[0] + p_ref[1]
    un = jnp.maximum(jnp.sqrt(jnp.sum(u * u, axis=1, keepdims=True)), MIN_NORM)
    g = jnp.tanh(un) * u / un
    gn = jnp.maximum(jnp.sqrt(jnp.sum(g * g, axis=1, keepdims=True)), MIN_NORM)
    maxnorm = 1.0 - EPS
    o_ref[...] = jnp.where(gn > maxnorm, g / gn * maxnorm, g)


_BR = RPT
_row_spec = pl.BlockSpec((_BR, D_FEAT), lambda i: (i, 0))
_pair_spec = pl.BlockSpec((NC, _BR, D_FEAT), lambda i: (0, i, 0))
_row_shape = jax.ShapeDtypeStruct((NP, D_FEAT), jnp.float32)

_logmap0 = pl.pallas_call(
    _logmap0_body, grid=(NP // _BR,),
    in_specs=[_row_spec], out_specs=_row_spec, out_shape=_row_shape)

_combine = pl.pallas_call(
    _combine_body, grid=(NP // _BR,),
    in_specs=[_pair_spec], out_specs=_row_spec, out_shape=_row_shape)

_finish = pl.pallas_call(
    _finish_body, grid=(NP // _BR,),
    in_specs=[_pair_spec], out_specs=_row_spec, out_shape=_row_shape)


def kernel(x, edge_index, edge_weight):
    src = edge_index[0].astype(jnp.int32)
    dst = edge_index[1].astype(jnp.int32)
    w = edge_weight.astype(jnp.float32)
    epad = E_PAD - N_EDGES
    src = jnp.concatenate([src, jnp.zeros((epad,), jnp.int32)])
    dst = jnp.concatenate([dst, jnp.zeros((epad,), jnp.int32)])
    wp = jnp.concatenate([w, jnp.zeros((epad,), jnp.float32)])
    xp = jnp.concatenate(
        [x, jnp.zeros((NP - N_NODES, D_FEAT), jnp.float32)])

    spmm = _make_spmm()
    t = _logmap0(xp)
    p1 = spmm(t, src, dst, wp)
    y1 = _combine(p1)
    p2 = spmm(y1, src, dst, wp)
    h = _finish(p2)
    return h[:N_NODES]


# speedup vs baseline: 4.8353x; 1.0909x over previous
import functools

import jax
import jax.numpy as jnp
from jax import lax
from jax.experimental import pallas as pl
from jax.experimental.pallas import tpu as pltpu
from jax.experimental.pallas import tpu_sc as plsc

N_NODES = 10000
D_FEAT = 128
N_EDGES = 320000
MIN_NORM = 1e-15
EPS = 4e-3

NC = 2
NS = 16
NW = NC * NS
K = 128
NB = -(-N_EDGES // (NW * K))
EPT = NB * K
E_PAD = EPT * NW
RPT = 632
NP = RPT * NS
DV = D_FEAT // 16


def _spmm_body(table, edata, out, accum,
               eb0, eb1, rows0, rows1, gsem0, gsem1, ssem0, ssem1):
    c = lax.axis_index("c")
    s = lax.axis_index("s")
    wid = c * NS + s

    zvec = jnp.zeros((16,), jnp.float32)

    def zero_row(i, carry):
        for d in range(DV):
            rows0[i, pl.ds(d * 16, 16)] = zvec
        return carry

    lax.fori_loop(0, K, zero_row, 0)
    r0 = s * RPT
    for j in range(RPT // K):
        pltpu.sync_copy(rows0, accum.at[pl.ds(r0 + j * K, K)])
    rem = RPT % K
    if rem:
        pltpu.sync_copy(rows0.at[pl.ds(0, rem)],
                        accum.at[pl.ds(r0 + (RPT // K) * K, rem)])
    plsc.subcore_barrier()

    base0 = wid * NB

    def stage(b, ebb):
        pltpu.sync_copy(edata.at[base0 + b], ebb)

    def gather(ebb, rows, sem):
        pltpu.async_copy(table.at[ebb.at[0]], rows, sem)

    def gwait(ebb, rows, sem):
        pltpu.make_async_copy(table.at[ebb.at[0]], rows, sem).wait()

    def scatter(ebb, rows, sem):
        pltpu.async_copy(rows, accum.at[ebb.at[1]], sem, add=True)

    def swait(ebb, rows, sem):
        pltpu.make_async_copy(rows, accum.at[ebb.at[1]], sem).wait()

    def scale(ebb, rows):
        def body(g, inner):
            w16 = lax.bitcast_convert_type(ebb[2, pl.ds(g * 16, 16)],
                                           jnp.float32)
            for l in range(16):
                wl = w16[l]
                for d in range(DV):
                    sl = pl.ds(d * 16, 16)
                    rows[g * 16 + l, sl] = rows[g * 16 + l, sl] * wl
            return inner

        lax.fori_loop(0, K // 16, body, 0)

    assert NB % 2 == 1
    stage(0, eb0)
    gather(eb0, rows0, gsem0)
    stage(1, eb1)
    gwait(eb0, rows0, gsem0)
    gather(eb1, rows1, gsem1)
    scale(eb0, rows0)
    scatter(eb0, rows0, ssem0)

    def pair(i, carry):
        b = 2 * i + 1
        swait(eb0, rows0, ssem0)
        stage(b + 1, eb0)
        gwait(eb1, rows1, gsem1)
        gather(eb0, rows0, gsem0)
        scale(eb1, rows1)
        scatter(eb1, rows1, ssem1)
        swait(eb1, rows1, ssem1)
        stage(b + 2, eb1)
        gwait(eb0, rows0, gsem0)
        gather(eb1, rows1, gsem1)
        scale(eb0, rows0)
        scatter(eb0, rows0, ssem0)
        return carry

    lax.fori_loop(0, (NB - 1) // 2, pair, 0)
    gwait(eb1, rows1, gsem1)
    swait(eb0, rows0, ssem0)
    plsc.subcore_barrier()

    pltpu.sync_copy(accum.at[pl.ds(r0, RPT)], out.at[c, pl.ds(r0, RPT)])


@functools.cache
def _make_spmm():
    return pl.kernel(
        _spmm_body,
        out_type=jax.ShapeDtypeStruct((NC, NP, D_FEAT), jnp.float32),
        mesh=plsc.VectorSubcoreMesh(core_axis_name="c", subcore_axis_name="s",
                                    num_cores=NC, num_subcores=NS),
        scratch_types=[
            pltpu.VMEM_SHARED((NP, D_FEAT), jnp.float32),
            pltpu.VMEM((3, K), jnp.int32),
            pltpu.VMEM((3, K), jnp.int32),
            pltpu.VMEM((K, D_FEAT), jnp.float32),
            pltpu.VMEM((K, D_FEAT), jnp.float32),
            pltpu.SemaphoreType.DMA,
            pltpu.SemaphoreType.DMA,
            pltpu.SemaphoreType.DMA,
            pltpu.SemaphoreType.DMA,
        ],
    )


def _logmap0_body(x_ref, o_ref):
    x = x_ref[...]
    norm = jnp.maximum(jnp.sqrt(jnp.sum(x * x, axis=1, keepdims=True)),
                       MIN_NORM)
    z = jnp.clip(norm, -1 + 1e-7, 1 - 1e-7)
    o_ref[...] = (0.5 * jnp.log((1 + z) / (1 - z)) / norm) * x


def _combine_body(p_ref, o_ref):
    o_ref[...] = p_ref[0] + p_ref[1]


def _finish_body(p_ref, o_ref):
    u = p_ref[0] + p_ref[1]
    un = jnp.maximum(jnp.sqrt(jnp.sum(u * u, axis=1, keepdims=True)), MIN_NORM)
    g = jnp.tanh(un) * u / un
    gn = jnp.maximum(jnp.sqrt(jnp.sum(g * g, axis=1, keepdims=True)), MIN_NORM)
    maxnorm = 1.0 - EPS
    o_ref[...] = jnp.where(gn > maxnorm, g / gn * maxnorm, g)


_BR = RPT
_row_spec = pl.BlockSpec((_BR, D_FEAT), lambda i: (i, 0))
_pair_spec = pl.BlockSpec((NC, _BR, D_FEAT), lambda i: (0, i, 0))
_row_shape = jax.ShapeDtypeStruct((NP, D_FEAT), jnp.float32)

_logmap0 = pl.pallas_call(
    _logmap0_body, grid=(NP // _BR,),
    in_specs=[_row_spec], out_specs=_row_spec, out_shape=_row_shape)

_combine = pl.pallas_call(
    _combine_body, grid=(NP // _BR,),
    in_specs=[_pair_spec], out_specs=_row_spec, out_shape=_row_shape)

_finish = pl.pallas_call(
    _finish_body, grid=(NP // _BR,),
    in_specs=[_pair_spec], out_specs=_row_spec, out_shape=_row_shape)


def kernel(x, edge_index, edge_weight):
    src = edge_index[0].astype(jnp.int32)
    dst = edge_index[1].astype(jnp.int32)
    w = edge_weight.astype(jnp.float32)
    epad = E_PAD - N_EDGES
    src = jnp.concatenate([src, jnp.zeros((epad,), jnp.int32)])
    dst = jnp.concatenate([dst, jnp.zeros((epad,), jnp.int32)])
    wbits = lax.bitcast_convert_type(
        jnp.concatenate([w, jnp.zeros((epad,), jnp.float32)]), jnp.int32)
    edata = (jnp.stack([src, dst, wbits], axis=0)
             .reshape(3, NW * NB, K).transpose(1, 0, 2))
    edata = jnp.concatenate([edata, jnp.zeros((1, 3, K), jnp.int32)])
    xp = jnp.concatenate(
        [x, jnp.zeros((NP - N_NODES, D_FEAT), jnp.float32)])

    spmm = _make_spmm()
    t = _logmap0(xp)
    p1 = spmm(t, edata)
    y1 = _combine(p1)
    p2 = spmm(y1, edata)
    h = _finish(p2)
    return h[:N_NODES]


# speedup vs baseline: 4.9618x; 1.0262x over previous
import functools

import jax
import jax.numpy as jnp
from jax import lax
from jax.experimental import pallas as pl
from jax.experimental.pallas import tpu as pltpu
from jax.experimental.pallas import tpu_sc as plsc

N_NODES = 10000
D_FEAT = 128
N_EDGES = 320000
MIN_NORM = 1e-15
EPS = 4e-3

NC = 2
NS = 16
NW = NC * NS
K = 128
NB = -(-N_EDGES // (NW * K))
EPT = NB * K
E_PAD = EPT * NW
RPT = 632
NP = RPT * NS
DV = D_FEAT // 16


def _spmm_body(table, edata, out, accum,
               eb0, eb1, rows0, rows1, gsem0, gsem1, ssem0, ssem1):
    c = lax.axis_index("c")
    s = lax.axis_index("s")
    wid = c * NS + s

    zvec = jnp.zeros((16,), jnp.float32)

    def zero_row(i, carry):
        for d in range(DV):
            rows0[i, pl.ds(d * 16, 16)] = zvec
        return carry

    lax.fori_loop(0, K, zero_row, 0)
    r0 = s * RPT
    for j in range(RPT // K):
        pltpu.sync_copy(rows0, accum.at[pl.ds(r0 + j * K, K)])
    rem = RPT % K
    if rem:
        pltpu.sync_copy(rows0.at[pl.ds(0, rem)],
                        accum.at[pl.ds(r0 + (RPT // K) * K, rem)])
    plsc.subcore_barrier()

    base0 = wid * NB

    def stage(b, ebb):
        pltpu.sync_copy(edata.at[base0 + b], ebb)

    def gather(ebb, rows, sem):
        pltpu.async_copy(table.at[ebb.at[0]], rows, sem)

    def gwait(ebb, rows, sem):
        pltpu.make_async_copy(table.at[ebb.at[0]], rows, sem).wait()

    def scatter(ebb, rows, sem):
        pltpu.async_copy(rows, accum.at[ebb.at[1]], sem, add=True)

    def swait(ebb, rows, sem):
        pltpu.make_async_copy(rows, accum.at[ebb.at[1]], sem).wait()

    def scale(ebb, rows):
        def body(g, inner):
            w16 = lax.bitcast_convert_type(ebb[2, pl.ds(g * 16, 16)],
                                           jnp.float32)
            for l in range(16):
                wl = w16[l]
                for d in range(DV):
                    sl = pl.ds(d * 16, 16)
                    rows[g * 16 + l, sl] = rows[g * 16 + l, sl] * wl
            return inner

        lax.fori_loop(0, K // 16, body, 0)

    assert NB % 2 == 1
    stage(0, eb0)
    gather(eb0, rows0, gsem0)
    stage(1, eb1)
    gwait(eb0, rows0, gsem0)
    gather(eb1, rows1, gsem1)
    scale(eb0, rows0)
    scatter(eb0, rows0, ssem0)

    def pair(i, carry):
        b = 2 * i + 1
        swait(eb0, rows0, ssem0)
        stage(b + 1, eb0)
        gather(eb0, rows0, gsem0)
        gwait(eb1, rows1, gsem1)
        scale(eb1, rows1)
        scatter(eb1, rows1, ssem1)
        swait(eb1, rows1, ssem1)
        stage(b + 2, eb1)
        gather(eb1, rows1, gsem1)
        gwait(eb0, rows0, gsem0)
        scale(eb0, rows0)
        scatter(eb0, rows0, ssem0)
        return carry

    lax.fori_loop(0, (NB - 1) // 2, pair, 0)
    gwait(eb1, rows1, gsem1)
    swait(eb0, rows0, ssem0)
    plsc.subcore_barrier()

    pltpu.sync_copy(accum.at[pl.ds(r0, RPT)], out.at[c, pl.ds(r0, RPT)])


@functools.cache
def _make_spmm():
    return pl.kernel(
        _spmm_body,
        out_type=jax.ShapeDtypeStruct((NC, NP, D_FEAT), jnp.float32),
        mesh=plsc.VectorSubcoreMesh(core_axis_name="c", subcore_axis_name="s",
                                    num_cores=NC, num_subcores=NS),
        scratch_types=[
            pltpu.VMEM_SHARED((NP, D_FEAT), jnp.float32),
            pltpu.VMEM((3, K), jnp.int32),
            pltpu.VMEM((3, K), jnp.int32),
            pltpu.VMEM((K, D_FEAT), jnp.float32),
            pltpu.VMEM((K, D_FEAT), jnp.float32),
            pltpu.SemaphoreType.DMA,
            pltpu.SemaphoreType.DMA,
            pltpu.SemaphoreType.DMA,
            pltpu.SemaphoreType.DMA,
        ],
    )


def _logmap0_body(x_ref, o_ref):
    x = x_ref[...]
    norm = jnp.maximum(jnp.sqrt(jnp.sum(x * x, axis=1, keepdims=True)),
                       MIN_NORM)
    z = jnp.clip(norm, -1 + 1e-7, 1 - 1e-7)
    o_ref[...] = (0.5 * jnp.log((1 + z) / (1 - z)) / norm) * x


def _combine_body(p_ref, o_ref):
    o_ref[...] = p_ref[0] + p_ref[1]


def _finish_body(p_ref, o_ref):
    u = p_ref[0] + p_ref[1]
    un = jnp.maximum(jnp.sqrt(jnp.sum(u * u, axis=1, keepdims=True)), MIN_NORM)
    g = jnp.tanh(un) * u / un
    gn = jnp.maximum(jnp.sqrt(jnp.sum(g * g, axis=1, keepdims=True)), MIN_NORM)
    maxnorm = 1.0 - EPS
    o_ref[...] = jnp.where(gn > maxnorm, g / gn * maxnorm, g)


_BR = RPT
_row_spec = pl.BlockSpec((_BR, D_FEAT), lambda i: (i, 0))
_pair_spec = pl.BlockSpec((NC, _BR, D_FEAT), lambda i: (0, i, 0))
_row_shape = jax.ShapeDtypeStruct((NP, D_FEAT), jnp.float32)

_logmap0 = pl.pallas_call(
    _logmap0_body, grid=(NP // _BR,),
    in_specs=[_row_spec], out_specs=_row_spec, out_shape=_row_shape)

_combine = pl.pallas_call(
    _combine_body, grid=(NP // _BR,),
    in_specs=[_pair_spec], out_specs=_row_spec, out_shape=_row_shape)

_finish = pl.pallas_call(
    _finish_body, grid=(NP // _BR,),
    in_specs=[_pair_spec], out_specs=_row_spec, out_shape=_row_shape)


def kernel(x, edge_index, edge_weight):
    src = edge_index[0].astype(jnp.int32)
    dst = edge_index[1].astype(jnp.int32)
    w = edge_weight.astype(jnp.float32)
    epad = E_PAD - N_EDGES
    src = jnp.concatenate([src, jnp.zeros((epad,), jnp.int32)])
    dst = jnp.concatenate([dst, jnp.zeros((epad,), jnp.int32)])
    wbits = lax.bitcast_convert_type(
        jnp.concatenate([w, jnp.zeros((epad,), jnp.float32)]), jnp.int32)
    edata = (jnp.stack([src, dst, wbits], axis=0)
             .reshape(3, NW * NB, K).transpose(1, 0, 2))
    edata = jnp.concatenate([edata, jnp.zeros((1, 3, K), jnp.int32)])
    xp = jnp.concatenate(
        [x, jnp.zeros((NP - N_NODES, D_FEAT), jnp.float32)])

    spmm = _make_spmm()
    t = _logmap0(xp)
    p1 = spmm(t, edata)
    y1 = _combine(p1)
    p2 = spmm(y1, edata)
    h = _finish(p2)
    return h[:N_NODES]


# speedup vs baseline: 10.6701x; 2.1505x over previous
import functools

import jax
import jax.numpy as jnp
from jax import lax
from jax.experimental import pallas as pl
from jax.experimental.pallas import tpu as pltpu
from jax.experimental.pallas import tpu_sc as plsc

N_NODES = 10000
D_FEAT = 128
N_EDGES = 320000
MIN_NORM = 1e-15
EPS = 4e-3

NC = 2
NS = 16
NW = NC * NS
K = 80
NB = 125
EPT = NB * K
E_PAD = EPT * NW
EROWS = NW * NB + 3
RPT = 632
DV = D_FEAT // 16
GPB = K // 16

assert NB % 4 == 1 and EPT >= N_EDGES // NW and RPT * (NS - 1) + 520 == N_NODES


def _spmm_body(table, edata, out, accum, *scr):
    ebs = scr[0:4]
    rbs = scr[4:8]
    gs = scr[8:12]
    ss = scr[12:16]
    ts = scr[16:20]
    c = lax.axis_index("c")
    s = lax.axis_index("s")
    wid = c * NS + s
    base0 = wid * NB
    r0 = s * RPT

    zvec = jnp.zeros((16,), jnp.float32)
    rb0 = rbs[0]

    def zero_row(i, carry):
        for d in range(DV):
            rb0[i, pl.ds(d * 16, 16)] = zvec
        return carry

    lax.fori_loop(0, K, zero_row, 0)

    @pl.when(s < NS - 1)
    def _():
        for j in range(RPT // K):
            pltpu.sync_copy(rb0, accum.at[pl.ds(r0 + j * K, K)])
        pltpu.sync_copy(rb0.at[pl.ds(0, RPT % K)],
                        accum.at[pl.ds(r0 + (RPT // K) * K, RPT % K)])

    @pl.when(s == NS - 1)
    def _():
        for j in range(520 // K):
            pltpu.sync_copy(rb0, accum.at[pl.ds(r0 + j * K, K)])
        pltpu.sync_copy(rb0.at[pl.ds(0, 520 % K)],
                        accum.at[pl.ds(r0 + (520 // K) * K, 520 % K)])

    plsc.subcore_barrier()

    def stage(b, j):
        pltpu.async_copy(edata.at[base0 + b], ebs[j], ts[j])

    def stwait(j):
        pltpu.make_async_copy(edata.at[base0], ebs[j], ts[j]).wait()

    def gather(j):
        pltpu.async_copy(table.at[ebs[j].at[0]], rbs[j], gs[j])

    def gwait(j):
        pltpu.make_async_copy(table.at[ebs[j].at[0]], rbs[j], gs[j]).wait()

    def scatter(j):
        pltpu.async_copy(rbs[j], accum.at[ebs[j].at[1]], ss[j], add=True)

    def swait(j):
        pltpu.make_async_copy(rbs[j], accum.at[ebs[j].at[1]], ss[j]).wait()

    def scale(j):
        ebb, rows = ebs[j], rbs[j]

        def body(g, inner):
            w16 = lax.bitcast_convert_type(ebb[2, pl.ds(g * 16, 16)],
                                           jnp.float32)
            for l in range(16):
                wl = w16[l]
                for d in range(DV):
                    sl = pl.ds(d * 16, 16)
                    rows[g * 16 + l, sl] = rows[g * 16 + l, sl] * wl
            return inner

        lax.fori_loop(0, GPB, body, 0)

    def step(b, j, first):
        gwait(j)
        scale(j)
        scatter(j)
        j1 = (j + 3) % 4
        if not first:
            swait(j1)
        stage(b + 3, j1)
        j2 = (j + 2) % 4
        stwait(j2)
        gather(j2)

    stage(0, 0)
    stage(1, 1)
    stage(2, 2)
    stwait(0)
    gather(0)
    stwait(1)
    gather(1)
    step(0, 0, first=True)

    def quad(i, carry):
        b = 4 * i + 1
        step(b, 1, False)
        step(b + 1, 2, False)
        step(b + 2, 3, False)
        step(b + 3, 0, False)
        return carry

    lax.fori_loop(0, (NB - 1) // 4, quad, 0)

    gwait((NB) % 4)
    gwait((NB + 1) % 4)
    swait((NB - 1) % 4)
    stwait((NB + 2) % 4)
    plsc.subcore_barrier()

    @pl.when(s < NS - 1)
    def _():
        pltpu.sync_copy(accum.at[pl.ds(r0, RPT)], out.at[c, pl.ds(r0, RPT)])

    @pl.when(s == NS - 1)
    def _():
        pltpu.sync_copy(accum.at[pl.ds(r0, 520)], out.at[c, pl.ds(r0, 520)])


@functools.cache
def _make_spmm():
    return pl.kernel(
        _spmm_body,
        out_type=jax.ShapeDtypeStruct((NC, N_NODES, D_FEAT), jnp.float32),
        mesh=plsc.VectorSubcoreMesh(core_axis_name="c", subcore_axis_name="s",
                                    num_cores=NC, num_subcores=NS),
        scratch_types=(
            [pltpu.VMEM_SHARED((N_NODES, D_FEAT), jnp.float32)]
            + [pltpu.VMEM((3, K), jnp.int32) for _ in range(4)]
            + [pltpu.VMEM((K, D_FEAT), jnp.float32) for _ in range(4)]
            + [pltpu.SemaphoreType.DMA for _ in range(12)]
        ),
    )


def _logmap0_body(x_ref, o_ref):
    x = x_ref[...]
    norm = jnp.maximum(jnp.sqrt(jnp.sum(x * x, axis=1, keepdims=True)),
                       MIN_NORM)
    z = jnp.clip(norm, -1 + 1e-7, 1 - 1e-7)
    o_ref[...] = (0.5 * jnp.log((1 + z) / (1 - z)) / norm) * x


def _combine_body(p_ref, o_ref):
    o_ref[...] = p_ref[0] + p_ref[1]


def _finish_body(p_ref, o_ref):
    u = p_ref[0] + p_ref[1]
    un = jnp.maximum(jnp.sqrt(jnp.sum(u * u, axis=1, keepdims=True)), MIN_NORM)
    g = jnp.tanh(un) * u / un
    gn = jnp.maximum(jnp.sqrt(jnp.sum(g * g, axis=1, keepdims=True)), MIN_NORM)
    maxnorm = 1.0 - EPS
    o_ref[...] = jnp.where(gn > maxnorm, g / gn * maxnorm, g)


_BR = 1000
_row_spec = pl.BlockSpec((_BR, D_FEAT), lambda i: (i, 0))
_pair_spec = pl.BlockSpec((NC, _BR, D_FEAT), lambda i: (0, i, 0))
_row_shape = jax.ShapeDtypeStruct((N_NODES, D_FEAT), jnp.float32)

_logmap0 = pl.pallas_call(
    _logmap0_body, grid=(N_NODES // _BR,),
    in_specs=[_row_spec], out_specs=_row_spec, out_shape=_row_shape)

_combine = pl.pallas_call(
    _combine_body, grid=(N_NODES // _BR,),
    in_specs=[_pair_spec], out_specs=_row_spec, out_shape=_row_shape)

_finish = pl.pallas_call(
    _finish_body, grid=(N_NODES // _BR,),
    in_specs=[_pair_spec], out_specs=_row_spec, out_shape=_row_shape)


def kernel(x, edge_index, edge_weight):
    src = edge_index[0].astype(jnp.int32)
    dst = edge_index[1].astype(jnp.int32)
    w = edge_weight.astype(jnp.float32)
    epad = E_PAD - N_EDGES
    src = jnp.concatenate([src, jnp.zeros((epad,), jnp.int32)])
    dst = jnp.concatenate([dst, jnp.zeros((epad,), jnp.int32)])
    wbits = lax.bitcast_convert_type(
        jnp.concatenate([w, jnp.zeros((epad,), jnp.float32)]), jnp.int32)
    edata = (jnp.stack([src, dst, wbits], axis=0)
             .reshape(3, NW * NB, K).transpose(1, 0, 2))
    edata = jnp.concatenate(
        [edata, jnp.zeros((EROWS - NW * NB, 3, K), jnp.int32)])

    spmm = _make_spmm()
    t = _logmap0(x)
    p1 = spmm(t, edata)
    y1 = _combine(p1)
    p2 = spmm(y1, edata)
    return _finish(p2)


# speedup vs baseline: 11.5412x; 1.0816x over previous
import functools

import jax
import jax.numpy as jnp
from jax import lax
from jax.experimental import pallas as pl
from jax.experimental.pallas import tpu as pltpu
from jax.experimental.pallas import tpu_sc as plsc

N_NODES = 10000
D_FEAT = 128
N_EDGES = 320000
MIN_NORM = 1e-15
EPS = 4e-3

NC = 2
NS = 16
NW = NC * NS
K = 80
NB = 125
NBR = NW * NB
RPT = 632
RPT_LAST = N_NODES - RPT * (NS - 1)
DV = D_FEAT // 16
GPB = K // 16

assert NB % 4 == 1 and NB * K * NW == N_EDGES and RPT_LAST % 8 == 0


def _spmm_body(table, srcb, dstb, wb, out, accum, *scr):
    sbs = scr[0:4]
    dbs = scr[4:8]
    wbs = scr[8:12]
    rbs = scr[12:16]
    gs = scr[16:20]
    ss = scr[20:24]
    ts = scr[24:28]
    c = lax.axis_index("c")
    s = lax.axis_index("s")
    wid = c * NS + s
    base0 = wid * NB
    r0 = s * RPT

    def stage(b, j):
        r = jnp.minimum(base0 + b, NBR - 1)
        pltpu.async_copy(srcb.at[r], sbs[j], ts[j])
        pltpu.async_copy(dstb.at[r], dbs[j], ts[j])
        pltpu.async_copy(wb.at[r], wbs[j], ts[j])

    def stwait(j):
        pltpu.make_async_copy(srcb.at[base0], sbs[j], ts[j]).wait()
        pltpu.make_async_copy(dstb.at[base0], dbs[j], ts[j]).wait()
        pltpu.make_async_copy(wb.at[base0], wbs[j], ts[j]).wait()

    def gather(j):
        pltpu.async_copy(table.at[sbs[j]], rbs[j], gs[j])

    def gwait(j):
        pltpu.make_async_copy(table.at[sbs[j]], rbs[j], gs[j]).wait()

    def scatter(j):
        pltpu.async_copy(rbs[j], accum.at[dbs[j]], ss[j], add=True)

    def swait(j):
        pltpu.make_async_copy(rbs[j], accum.at[dbs[j]], ss[j]).wait()

    def scale(j):
        wv, rows = wbs[j], rbs[j]

        def body(g, inner):
            w16 = wv[pl.ds(g * 16, 16)]
            for l in range(16):
                wl = w16[l]
                for d in range(DV):
                    sl = pl.ds(d * 16, 16)
                    rows[g * 16 + l, sl] = rows[g * 16 + l, sl] * wl
            return inner

        lax.fori_loop(0, GPB, body, 0)

    stage(0, 0)
    stage(1, 1)
    stage(2, 2)
    stwait(0)
    gather(0)
    stwait(1)
    gather(1)

    zvec = jnp.zeros((16,), jnp.float32)
    zb = rbs[3]

    def zero_row(i, carry):
        for d in range(DV):
            zb[i, pl.ds(d * 16, 16)] = zvec
        return carry

    lax.fori_loop(0, K, zero_row, 0)

    @pl.when(s < NS - 1)
    def _():
        for j in range(RPT // K):
            pltpu.sync_copy(zb, accum.at[pl.ds(r0 + j * K, K)])
        pltpu.sync_copy(zb.at[pl.ds(0, RPT % K)],
                        accum.at[pl.ds(r0 + (RPT // K) * K, RPT % K)])

    @pl.when(s == NS - 1)
    def _():
        for j in range(RPT_LAST // K):
            pltpu.sync_copy(zb, accum.at[pl.ds(r0 + j * K, K)])
        pltpu.sync_copy(zb.at[pl.ds(0, RPT_LAST % K)],
                        accum.at[pl.ds(r0 + (RPT_LAST // K) * K,
                                       RPT_LAST % K)])

    plsc.subcore_barrier()

    def step(b, j, first):
        gwait(j)
        scale(j)
        scatter(j)
        j1 = (j + 3) % 4
        if not first:
            swait(j1)
        stage(b + 3, j1)
        j2 = (j + 2) % 4
        stwait(j2)
        gather(j2)

    step(0, 0, first=True)

    def quad(i, carry):
        b = 4 * i + 1
        step(b, 1, False)
        step(b + 1, 2, False)
        step(b + 2, 3, False)
        step(b + 3, 0, False)
        return carry

    lax.fori_loop(0, (NB - 1) // 4, quad, 0)

    gwait(NB % 4)
    gwait((NB + 1) % 4)
    swait((NB - 1) % 4)
    stwait((NB + 2) % 4)
    plsc.subcore_barrier()

    @pl.when(s < NS - 1)
    def _():
        pltpu.sync_copy(accum.at[pl.ds(r0, RPT)], out.at[c, pl.ds(r0, RPT)])

    @pl.when(s == NS - 1)
    def _():
        pltpu.sync_copy(accum.at[pl.ds(r0, RPT_LAST)],
                        out.at[c, pl.ds(r0, RPT_LAST)])


@functools.cache
def _make_spmm():
    return pl.kernel(
        _spmm_body,
        out_type=jax.ShapeDtypeStruct((NC, N_NODES, D_FEAT), jnp.float32),
        mesh=plsc.VectorSubcoreMesh(core_axis_name="c", subcore_axis_name="s",
                                    num_cores=NC, num_subcores=NS),
        scratch_types=(
            [pltpu.VMEM_SHARED((N_NODES, D_FEAT), jnp.float32)]
            + [pltpu.VMEM((K,), jnp.int32) for _ in range(8)]
            + [pltpu.VMEM((K,), jnp.float32) for _ in range(4)]
            + [pltpu.VMEM((K, D_FEAT), jnp.float32) for _ in range(4)]
            + [pltpu.SemaphoreType.DMA for _ in range(12)]
        ),
    )


def _spmm(table, srcb, dstb, wb):
    return _make_spmm()(table, srcb, dstb, wb)


def _logmap0_body(x_ref, o_ref):
    x = x_ref[...]
    norm = jnp.maximum(jnp.sqrt(jnp.sum(x * x, axis=1, keepdims=True)),
                       MIN_NORM)
    z = jnp.clip(norm, -1 + 1e-7, 1 - 1e-7)
    o_ref[...] = (0.5 * jnp.log((1 + z) / (1 - z)) / norm) * x


def _combine_body(p_ref, o_ref):
    o_ref[...] = p_ref[0] + p_ref[1]


def _finish_body(p_ref, o_ref):
    u = p_ref[0] + p_ref[1]
    un = jnp.maximum(jnp.sqrt(jnp.sum(u * u, axis=1, keepdims=True)), MIN_NORM)
    g = jnp.tanh(un) * u / un
    gn = jnp.maximum(jnp.sqrt(jnp.sum(g * g, axis=1, keepdims=True)), MIN_NORM)
    maxnorm = 1.0 - EPS
    o_ref[...] = jnp.where(gn > maxnorm, g / gn * maxnorm, g)


_BR = 1000
_row_spec = pl.BlockSpec((_BR, D_FEAT), lambda i: (i, 0))
_pair_spec = pl.BlockSpec((NC, _BR, D_FEAT), lambda i: (0, i, 0))
_row_shape = jax.ShapeDtypeStruct((N_NODES, D_FEAT), jnp.float32)

_logmap0 = pl.pallas_call(
    _logmap0_body, grid=(N_NODES // _BR,),
    in_specs=[_row_spec], out_specs=_row_spec, out_shape=_row_shape)

_combine = pl.pallas_call(
    _combine_body, grid=(N_NODES // _BR,),
    in_specs=[_pair_spec], out_specs=_row_spec, out_shape=_row_shape)

_finish = pl.pallas_call(
    _finish_body, grid=(N_NODES // _BR,),
    in_specs=[_pair_spec], out_specs=_row_spec, out_shape=_row_shape)


def kernel(x, edge_index, edge_weight):
    srcb = edge_index[0].astype(jnp.int32).reshape(NBR, K)
    dstb = edge_index[1].astype(jnp.int32).reshape(NBR, K)
    wb = edge_weight.astype(jnp.float32).reshape(NBR, K)

    t = _logmap0(x)
    p1 = _spmm(t, srcb, dstb, wb)
    y1 = _combine(p1)
    p2 = _spmm(y1, srcb, dstb, wb)
    return _finish(p2)
